# baseline (speedup 1.0000x reference)
import functools

import jax
import jax.numpy as jnp
from jax import lax
from jax.experimental import pallas as pl
from jax.experimental.pallas import tpu as pltpu
from jax.experimental.pallas import tpu_sc as plsc

N_ATOMS_K = 10000
N_PAIRS_K = 160000
XDIM = 3
N_PROP_K = 128
ROWS = N_ATOMS_K * XDIM

TC_BLK = 1000
TC_GRID = N_ATOMS_K // TC_BLK

L = 16


def _proj_body(px_ref, wi_ref, wj_ref, w0_ref, w1_ref, yi_ref, yj_ref,
               a_scr, b_scr):
    @pl.when(pl.program_id(0) == 0)
    def _():
        w01 = jnp.dot(w0_ref[...], w1_ref[...],
                      preferred_element_type=jnp.float32,
                      precision=lax.Precision.HIGHEST)
        a_scr[...] = jnp.dot(wi_ref[...], w01,
                             preferred_element_type=jnp.float32,
                             precision=lax.Precision.HIGHEST)
        b_scr[...] = jnp.dot(wj_ref[...], w01,
                             preferred_element_type=jnp.float32,
                             precision=lax.Precision.HIGHEST)

    x = px_ref[...].reshape(TC_BLK * XDIM, N_PROP_K)
    yi_ref[...] = jnp.dot(
        x, a_scr[...],
        preferred_element_type=jnp.float32).reshape(TC_BLK, XDIM, N_PROP_K)
    yj_ref[...] = jnp.dot(
        x, b_scr[...],
        preferred_element_type=jnp.float32).reshape(TC_BLK, XDIM, N_PROP_K)


@jax.jit
def _project(px, Wi, Wj, W0, W1):
    wspec = pl.BlockSpec((N_PROP_K, N_PROP_K), lambda i: (0, 0))
    tspec = pl.BlockSpec((TC_BLK, XDIM, N_PROP_K), lambda i: (i, 0, 0))
    return pl.pallas_call(
        _proj_body,
        grid=(TC_GRID,),
        in_specs=[tspec, wspec, wspec, wspec, wspec],
        out_specs=[tspec, tspec],
        out_shape=[jax.ShapeDtypeStruct((N_ATOMS_K, XDIM, N_PROP_K),
                                        jnp.float32)] * 2,
        scratch_shapes=[
            pltpu.VMEM((N_PROP_K, N_PROP_K), jnp.float32),
            pltpu.VMEM((N_PROP_K, N_PROP_K), jnp.float32),
        ],
    )(px, Wi, Wj, W0, W1)


def _make_sc_gather(n_pairs, chunk):
    info = plsc.get_sparse_core_info()
    nc, ns = info.num_cores, info.num_subcores
    nw = nc * ns
    per_w = n_pairs // nw
    n_real = per_w // chunk
    assert n_real * chunk == per_w
    n_chunks = n_real + 1 if n_real % 2 else n_real + 2

    mesh = plsc.VectorSubcoreMesh(core_axis_name="c", subcore_axis_name="s")

    buf_t = pltpu.VMEM((chunk, XDIM, N_PROP_K), jnp.float32)

    @functools.partial(
        pl.kernel,
        mesh=mesh,
        out_type=jax.ShapeDtypeStruct((n_pairs, XDIM, N_PROP_K),
                                      jnp.float32),
        scratch_types=[
            pltpu.VMEM((n_chunks, chunk), jnp.int32),
            pltpu.VMEM((n_chunks, chunk), jnp.int32),
            buf_t, buf_t, buf_t, buf_t,
            pltpu.SemaphoreType.DMA, pltpu.SemaphoreType.DMA,
            pltpu.SemaphoreType.DMA, pltpu.SemaphoreType.DMA,
            pltpu.SemaphoreType.DMA, pltpu.SemaphoreType.DMA,
        ],
    )
    def sc_gather(yi_hbm, yj_hbm, idxi_hbm, idxj_hbm, out_hbm,
                  idxi_v, idxj_v, ga0, ga1, gb0, gb1,
                  gsa0, gsa1, gsb0, gsb1, sts0, sts1):
        wid = lax.axis_index("s") * nc + lax.axis_index("c")
        base = wid * per_w
        ga = (ga0, ga1)
        gb = (gb0, gb1)
        gsa = (gsa0, gsa1)
        gsb = (gsb0, gsb1)
        sts = (sts0, sts1)
        pltpu.sync_copy(idxi_hbm.at[wid], idxi_v)
        pltpu.sync_copy(idxj_hbm.at[wid], idxj_v)

        half = chunk // 2

        def _gather_parts(c, par):
            return (
                (yi_hbm.at[idxi_v.at[c, pl.ds(0, half)]],
                 ga[par].at[pl.ds(0, half)], gsa[par]),
                (yi_hbm.at[idxi_v.at[c, pl.ds(half, half)]],
                 ga[par].at[pl.ds(half, half)], gsa[par]),
                (yj_hbm.at[idxj_v.at[c, pl.ds(0, half)]],
                 gb[par].at[pl.ds(0, half)], gsb[par]),
                (yj_hbm.at[idxj_v.at[c, pl.ds(half, half)]],
                 gb[par].at[pl.ds(half, half)], gsb[par]),
            )

        def issue_gather(c, par):
            for src, dst, sem in _gather_parts(c, par):
                pltpu.async_copy(src, dst, sem)

        def wait_gather(c, par):
            for src, dst, sem in _gather_parts(c, par):
                pltpu.make_async_copy(src, dst, sem).wait()

        def wait_store(c, par):
            pltpu.make_async_copy(
                ga[par], out_hbm.at[pl.ds(base + c * chunk, chunk)],
                sts[par]).wait()

        issue_gather(0, 0)

        def step(s, carry):
            for b in range(2):
                c = 2 * s + b
                par = b
                opar = 1 - b

                wait_gather(c, par)

                @pl.when(c <= n_real - 1)
                def _():
                    def row_body(r, cr):
                        for x in range(XDIM):
                            for dd in range(N_PROP_K // L):
                                sl = pl.ds(dd * L, L)
                                ga[par][r, x, sl] = (
                                    ga[par][r, x, sl] + gb[par][r, x, sl])
                        return cr

                    lax.fori_loop(0, chunk, row_body, 0)

                @pl.when((c >= 1) & (c <= n_real))
                def _():
                    wait_store(c - 1, opar)

                @pl.when(c + 1 <= n_chunks - 1)
                def _():
                    issue_gather(c + 1, opar)

                @pl.when(c <= n_real - 1)
                def _():
                    pltpu.async_copy(
                        ga[par],
                        out_hbm.at[pl.ds(base + c * chunk, chunk)],
                        sts[par])
            return carry

        lax.fori_loop(0, n_chunks // 2, step, 0)

    return sc_gather, nw, per_w, n_chunks, chunk


N_SPLIT = 2


def kernel(ind_2, px, Wi, Wj, W0, W1):
    half_pairs = N_PAIRS_K // N_SPLIT
    sc_gather, nw, per_w, n_chunks, chunk = _make_sc_gather(
        half_pairs, 50 if (half_pairs // 32) % 40 else 40)

    yi, yj = _project(px, Wi, Wj, W0, W1)

    ind = ind_2.astype(jnp.int32)
    pad = n_chunks * chunk - per_w

    def prep(col):
        a = col.reshape(nw, per_w)
        a = jnp.pad(a, ((0, 0), (0, pad)))
        return a.reshape(nw, n_chunks, chunk)

    outs = []
    for h in range(N_SPLIT):
        sl = ind[h * half_pairs:(h + 1) * half_pairs]
        outs.append(sc_gather(yi, yj, prep(sl[:, 0]), prep(sl[:, 1])))
    return jnp.concatenate(outs, axis=0)


# speedup vs baseline: 1.6182x; 1.6182x over previous
import functools

import jax
import jax.numpy as jnp
from jax import lax
from jax.experimental import pallas as pl
from jax.experimental.pallas import tpu as pltpu
from jax.experimental.pallas import tpu_sc as plsc

N_ATOMS_K = 10000
N_PAIRS_K = 160000
XDIM = 3
N_PROP_K = 128
ROWS = N_ATOMS_K * XDIM

TC_BLK = 1000
TC_GRID = N_ATOMS_K // TC_BLK

L = 16


def _proj_body(px_ref, wi_ref, wj_ref, w0_ref, w1_ref, yi_ref, yj_ref,
               a_scr, b_scr):
    @pl.when(pl.program_id(0) == 0)
    def _():
        w01 = jnp.dot(w0_ref[...], w1_ref[...],
                      preferred_element_type=jnp.float32,
                      precision=lax.Precision.HIGHEST)
        a_scr[...] = jnp.dot(wi_ref[...], w01,
                             preferred_element_type=jnp.float32,
                             precision=lax.Precision.HIGHEST)
        b_scr[...] = jnp.dot(wj_ref[...], w01,
                             preferred_element_type=jnp.float32,
                             precision=lax.Precision.HIGHEST)

    x = px_ref[...].reshape(TC_BLK * XDIM, N_PROP_K)
    yi_ref[...] = jnp.dot(
        x, a_scr[...],
        preferred_element_type=jnp.float32).reshape(TC_BLK, XDIM, N_PROP_K)
    yj_ref[...] = jnp.dot(
        x, b_scr[...],
        preferred_element_type=jnp.float32).reshape(TC_BLK, XDIM, N_PROP_K)


@jax.jit
def _project(px, Wi, Wj, W0, W1):
    wspec = pl.BlockSpec((N_PROP_K, N_PROP_K), lambda i: (0, 0))
    tspec = pl.BlockSpec((TC_BLK, XDIM, N_PROP_K), lambda i: (i, 0, 0))
    return pl.pallas_call(
        _proj_body,
        grid=(TC_GRID,),
        in_specs=[tspec, wspec, wspec, wspec, wspec],
        out_specs=[tspec, tspec],
        out_shape=[jax.ShapeDtypeStruct((N_ATOMS_K, XDIM, N_PROP_K),
                                        jnp.float32)] * 2,
        scratch_shapes=[
            pltpu.VMEM((N_PROP_K, N_PROP_K), jnp.float32),
            pltpu.VMEM((N_PROP_K, N_PROP_K), jnp.float32),
        ],
    )(px, Wi, Wj, W0, W1)


def _make_sc_gather(n_pairs, chunk):
    info = plsc.get_sparse_core_info()
    nc, ns = info.num_cores, info.num_subcores
    nw = nc * ns
    per_w = n_pairs // nw
    n_real = per_w // chunk
    assert n_real * chunk == per_w
    n_chunks = n_real + 1 if n_real % 2 else n_real + 2

    mesh = plsc.VectorSubcoreMesh(core_axis_name="c", subcore_axis_name="s")

    buf_t = pltpu.VMEM((chunk, XDIM, N_PROP_K), jnp.float32)

    @functools.partial(
        pl.kernel,
        mesh=mesh,
        out_type=jax.ShapeDtypeStruct((n_pairs, XDIM, N_PROP_K),
                                      jnp.float32),
        scratch_types=[
            pltpu.VMEM((n_chunks, chunk), jnp.int32),
            pltpu.VMEM((n_chunks, chunk), jnp.int32),
            buf_t, buf_t, buf_t, buf_t,
            pltpu.SemaphoreType.DMA, pltpu.SemaphoreType.DMA,
            pltpu.SemaphoreType.DMA, pltpu.SemaphoreType.DMA,
            pltpu.SemaphoreType.DMA, pltpu.SemaphoreType.DMA,
        ],
    )
    def sc_gather(yi_hbm, yj_hbm, idxi_hbm, idxj_hbm, out_hbm,
                  idxi_v, idxj_v, ga0, ga1, gb0, gb1,
                  gsa0, gsa1, gsb0, gsb1, sts0, sts1):
        wid = lax.axis_index("s") * nc + lax.axis_index("c")
        base = wid * per_w
        ga = (ga0, ga1)
        gb = (gb0, gb1)
        gsa = (gsa0, gsa1)
        gsb = (gsb0, gsb1)
        sts = (sts0, sts1)
        pltpu.sync_copy(idxi_hbm.at[wid], idxi_v)
        pltpu.sync_copy(idxj_hbm.at[wid], idxj_v)

        half = chunk // 2

        def _gather_parts(c, par):
            return (
                (yi_hbm.at[idxi_v.at[c, pl.ds(0, half)]],
                 ga[par].at[pl.ds(0, half)], gsa[par]),
                (yi_hbm.at[idxi_v.at[c, pl.ds(half, half)]],
                 ga[par].at[pl.ds(half, half)], gsa[par]),
                (yj_hbm.at[idxj_v.at[c, pl.ds(0, half)]],
                 gb[par].at[pl.ds(0, half)], gsb[par]),
                (yj_hbm.at[idxj_v.at[c, pl.ds(half, half)]],
                 gb[par].at[pl.ds(half, half)], gsb[par]),
            )

        def issue_gather(c, par):
            for src, dst, sem in _gather_parts(c, par):
                pltpu.async_copy(src, dst, sem)

        def wait_gather(c, par):
            for src, dst, sem in _gather_parts(c, par):
                pltpu.make_async_copy(src, dst, sem).wait()

        def wait_store(c, par):
            pltpu.make_async_copy(
                ga[par], out_hbm.at[pl.ds(base + c * chunk, chunk)],
                sts[par]).wait()

        issue_gather(0, 0)

        def step(s, carry):
            for b in range(2):
                c = 2 * s + b
                par = b
                opar = 1 - b

                wait_gather(c, par)

                @pl.when(c <= n_real - 1)
                def _():
                    def row_body(r, cr):
                        for x in range(XDIM):
                            for dd in range(N_PROP_K // L):
                                sl = pl.ds(dd * L, L)
                                ga[par][r, x, sl] = (
                                    ga[par][r, x, sl] + gb[par][r, x, sl])
                        return cr

                    lax.fori_loop(0, chunk, row_body, 0)

                @pl.when((c >= 1) & (c <= n_real))
                def _():
                    wait_store(c - 1, opar)

                @pl.when(c + 1 <= n_chunks - 1)
                def _():
                    issue_gather(c + 1, opar)

                @pl.when(c <= n_real - 1)
                def _():
                    pltpu.async_copy(
                        ga[par],
                        out_hbm.at[pl.ds(base + c * chunk, chunk)],
                        sts[par])
            return carry

        lax.fori_loop(0, n_chunks // 2, step, 0)

    return sc_gather, nw, per_w, n_chunks, chunk


def kernel(ind_2, px, Wi, Wj, W0, W1):
    sc_gather, nw, per_w, n_chunks, chunk = _make_sc_gather(N_PAIRS_K, 40)

    yi, yj = _project(px, Wi, Wj, W0, W1)

    ind = ind_2.astype(jnp.int32)
    pad = n_chunks * chunk - per_w

    def prep(col):
        a = col.reshape(nw, per_w)
        a = jnp.pad(a, ((0, 0), (0, pad)))
        return a.reshape(nw, n_chunks, chunk)

    idxi = prep(ind[:, 0])
    idxj = prep(ind[:, 1])

    return sc_gather(yi, yj, idxi, idxj)
